# Initial kernel scaffold; baseline (speedup 1.0000x reference)
import jax
import jax.numpy as jnp
from jax.experimental import pallas as pl


def kernel(pf_points, pf_features, sv_points, sv_features, pf_batch, sv_batch, params):
    raise NotImplementedError("write your pallas kernel here")


# speedup vs baseline: 4.2391x; 4.2391x over previous
import functools

import jax
import jax.numpy as jnp
from jax.experimental import pallas as pl
from jax.experimental.pallas import tpu as pltpu

F32 = jnp.float32
EPS = 1e-5


def _pick_div(n, cap):
    best = None
    for d in range(1, min(n, cap) + 1):
        if n % d == 0 and d % 8 == 0:
            best = d
    if best is None:
        best = n if n <= cap else 8
    return best


def _knn_body(dstx_ref, srcx_ref, srcn_ref, dstb_ref, srcb_ref, idx_ref,
              valid_ref, *, k, ns, bd):
    dst = dstx_ref[...]
    src = srcx_ref[...]
    dstn = jnp.sum(dst * dst, axis=1, keepdims=True)
    dot = jax.lax.dot_general(dst, src, (((1,), (1,)), ((), ())),
                              preferred_element_type=F32)
    d2 = (dstn - 2.0 * dot) + srcn_ref[...]
    d2 = jnp.where(dstb_ref[...] != srcb_ref[...], jnp.float32(1e18), d2)
    col = jax.lax.broadcasted_iota(jnp.int32, (bd, ns), 1)
    idx_cols, val_cols = [], []
    for t in range(k):
        m = jnp.min(d2, axis=1, keepdims=True)
        am = jnp.min(jnp.where(d2 <= m, col, ns), axis=1, keepdims=True)
        idx_cols.append(am)
        val_cols.append((m < 1e17).astype(F32))
        if t + 1 < k:
            d2 = jnp.where(col == am, jnp.float32(3e18), d2)
    idx_ref[...] = jnp.concatenate(idx_cols, axis=1) if k > 1 else idx_cols[0]
    valid_ref[...] = (jnp.concatenate(val_cols, axis=1) if k > 1
                      else val_cols[0])


def _rownorm_body(x_ref, y_ref):
    x = x_ref[...]
    y_ref[...] = jnp.sum(x * x, axis=1, keepdims=True)


def _rownorm_call(x):
    n = x.shape[0]
    return pl.pallas_call(
        _rownorm_body,
        out_shape=jax.ShapeDtypeStruct((n, 1), F32),
    )(x)


def _knn_call(dst_x, src_x, dst_b, src_b, k):
    nd, dim = dst_x.shape
    ns = src_x.shape[0]
    bd = _pick_div(nd, 512)
    grid = (nd // bd,)
    srcn_row = _rownorm_call(src_x).reshape(1, ns)
    body = functools.partial(_knn_body, k=k, ns=ns, bd=bd)
    idx, valid = pl.pallas_call(
        body,
        grid=grid,
        in_specs=[pl.BlockSpec((bd, dim), lambda i: (i, 0)),
                  pl.BlockSpec((ns, dim), lambda i: (0, 0)),
                  pl.BlockSpec((1, ns), lambda i: (0, 0)),
                  pl.BlockSpec((bd, 1), lambda i: (i, 0)),
                  pl.BlockSpec((1, ns), lambda i: (0, 0))],
        out_specs=[pl.BlockSpec((bd, k), lambda i: (i, 0)),
                   pl.BlockSpec((bd, k), lambda i: (i, 0))],
        out_shape=[jax.ShapeDtypeStruct((nd, k), jnp.int32),
                   jax.ShapeDtypeStruct((nd, k), F32)],
    )(dst_x, src_x, srcn_row, dst_b.reshape(nd, 1), src_b.reshape(1, ns))
    return idx, valid


def _mm_body(x_ref, w_ref, b_ref, y_ref):
    y_ref[...] = (jnp.dot(x_ref[...], w_ref[...], preferred_element_type=F32)
                  + b_ref[...])


def _mm_call(x, w, b):
    n, dout = x.shape[0], w.shape[1]
    if b is None:
        b = jnp.zeros((1, dout), F32)
    else:
        b = b.reshape(1, dout)
    return pl.pallas_call(
        _mm_body,
        out_shape=jax.ShapeDtypeStruct((n, dout), F32),
    )(x, w, b)


def _bn_body(x_ref, g_ref, bt_ref, y_ref, *, n):
    x = x_ref[...]
    m = jnp.sum(x, 0, keepdims=True) / n
    d = x - m
    v = jnp.sum(d * d, 0, keepdims=True) / n
    y_ref[...] = g_ref[...] * d * jax.lax.rsqrt(v + EPS) + bt_ref[...]


def _bn_call(x, g, bt):
    n, dim = x.shape
    return pl.pallas_call(
        functools.partial(_bn_body, n=float(n)),
        out_shape=jax.ShapeDtypeStruct((n, dim), F32),
    )(x, g.reshape(1, dim), bt.reshape(1, dim))


def _fusion_body(x_ref, w_ref, g_ref, bt_ref, y_ref, *, n):
    z = jnp.dot(x_ref[...], w_ref[...], preferred_element_type=F32)
    m = jnp.sum(z, 0, keepdims=True) / n
    d = z - m
    v = jnp.sum(d * d, 0, keepdims=True) / n
    y_ref[...] = jnp.maximum(
        g_ref[...] * d * jax.lax.rsqrt(v + EPS) + bt_ref[...], 0.0)


def _fusion_call(x, w, g, bt):
    n = x.shape[0]
    dout = w.shape[1]
    return pl.pallas_call(
        functools.partial(_fusion_body, n=float(n)),
        out_shape=jax.ShapeDtypeStruct((n, dout), F32),
    )(x, w, g.reshape(1, dout), bt.reshape(1, dout))


def _l1_body(idx_sref, a_ref, b_ref, h_ref, st_ref, *, k, bd, grid_i, d1):
    j = pl.program_id(0)
    i = pl.program_id(1)
    base = i * bd

    def body(d, carry):
        si = idx_sref[j, base + d]
        h_ref[pl.ds(d, 1), :] = a_ref[pl.ds(d, 1), :] + b_ref[pl.ds(si, 1), :]
        return carry

    jax.lax.fori_loop(0, bd, body, 0, unroll=8)
    h = h_ref[...]
    s = jnp.concatenate(
        [jnp.sum(h, 0, keepdims=True), jnp.sum(h * h, 0, keepdims=True),
         jnp.zeros((6, d1), F32)], axis=0)

    @pl.when((j == 0) & (i == 0))
    def _():
        st_ref[...] = jnp.zeros_like(st_ref)

    st_ref[...] += s


def _l1_call(idx_t, a, b, k):
    nd, d1 = a.shape
    ns = b.shape[0]
    bd = _pick_div(nd, 512)
    gi = nd // bd
    body = functools.partial(_l1_body, k=k, bd=bd, grid_i=gi, d1=d1)
    grid_spec = pltpu.PrefetchScalarGridSpec(
        num_scalar_prefetch=1,
        grid=(k, gi),
        in_specs=[pl.BlockSpec((bd, d1), lambda j, i, *_: (i, 0)),
                  pl.BlockSpec((ns, d1), lambda j, i, *_: (0, 0))],
        out_specs=[pl.BlockSpec((bd, d1), lambda j, i, *_: (j * gi + i, 0)),
                   pl.BlockSpec((8, d1), lambda j, i, *_: (0, 0))],
    )
    h, st = pl.pallas_call(
        body,
        grid_spec=grid_spec,
        out_shape=[jax.ShapeDtypeStruct((k * nd, d1), F32),
                   jax.ShapeDtypeStruct((8, d1), F32)],
    )(idx_t, a, b)
    return h, st


def _dense_body(stin_ref, x_ref, g_ref, bt_ref, w_ref, b_ref, y_ref, st_ref,
                *, n_prev, dout):
    st = stin_ref[...]
    mean = st[0:1, :] * (1.0 / n_prev)
    var = jnp.maximum(st[1:2, :] * (1.0 / n_prev) - mean * mean, 0.0)
    scale = g_ref[...] * jax.lax.rsqrt(var + EPS)
    shift = bt_ref[...] - scale * mean
    xn = jnp.maximum(x_ref[...] * scale + shift, 0.0)
    y = (jnp.dot(xn, w_ref[...], preferred_element_type=F32) + b_ref[...])
    y_ref[...] = y
    s = jnp.concatenate(
        [jnp.sum(y, 0, keepdims=True), jnp.sum(y * y, 0, keepdims=True),
         jnp.zeros((6, dout), F32)], axis=0)

    @pl.when(pl.program_id(0) == 0)
    def _():
        st_ref[...] = jnp.zeros_like(st_ref)

    st_ref[...] += s


def _dense_call(h, st, g, bt, w, b, n_prev):
    ne, din = h.shape
    dout = w.shape[1]
    be = _pick_div(ne, 2048)
    body = functools.partial(_dense_body, n_prev=float(n_prev), dout=dout)
    y, st_out = pl.pallas_call(
        body,
        grid=(ne // be,),
        in_specs=[pl.BlockSpec((8, din), lambda i: (0, 0)),
                  pl.BlockSpec((be, din), lambda i: (i, 0)),
                  pl.BlockSpec((1, din), lambda i: (0, 0)),
                  pl.BlockSpec((1, din), lambda i: (0, 0)),
                  pl.BlockSpec((din, dout), lambda i: (0, 0)),
                  pl.BlockSpec((1, dout), lambda i: (0, 0))],
        out_specs=[pl.BlockSpec((be, dout), lambda i: (i, 0)),
                   pl.BlockSpec((8, dout), lambda i: (0, 0))],
        out_shape=[jax.ShapeDtypeStruct((ne, dout), F32),
                   jax.ShapeDtypeStruct((8, dout), F32)],
    )(st, h, g.reshape(1, din), bt.reshape(1, din), w, b.reshape(1, dout))
    return y, st_out


def _agg_body(stin_ref, g_ref, bt_ref, h_ref, v_ref, out_ref, acc_ref,
              cnt_ref, *, n_e, k):
    j = pl.program_id(1)
    st = stin_ref[...]
    mean = st[0:1, :] * (1.0 / n_e)
    var = jnp.maximum(st[1:2, :] * (1.0 / n_e) - mean * mean, 0.0)
    scale = g_ref[...] * jax.lax.rsqrt(var + EPS)
    shift = bt_ref[...] - scale * mean
    m = jnp.maximum(h_ref[...] * scale + shift, 0.0)
    w = v_ref[...]

    @pl.when(j == 0)
    def _():
        acc_ref[...] = jnp.zeros_like(acc_ref)
        cnt_ref[...] = jnp.zeros_like(cnt_ref)

    acc_ref[...] += m * w
    cnt_ref[...] += w

    @pl.when(j == k - 1)
    def _():
        out_ref[...] = acc_ref[...] / jnp.maximum(cnt_ref[...], 1.0)


def _agg_call(h, st, g, bt, valid_e, k, nd):
    ne, d = h.shape
    bd = _pick_div(nd, 1024)
    gi = nd // bd
    body = functools.partial(_agg_body, n_e=float(ne), k=k)
    return pl.pallas_call(
        body,
        grid=(gi, k),
        in_specs=[pl.BlockSpec((8, d), lambda i, j: (0, 0)),
                  pl.BlockSpec((1, d), lambda i, j: (0, 0)),
                  pl.BlockSpec((1, d), lambda i, j: (0, 0)),
                  pl.BlockSpec((bd, d), lambda i, j: (j * gi + i, 0)),
                  pl.BlockSpec((bd, 1), lambda i, j: (j * gi + i, 0))],
        out_specs=pl.BlockSpec((bd, d), lambda i, j: (i, 0)),
        out_shape=jax.ShapeDtypeStruct((nd, d), F32),
        scratch_shapes=[pltpu.VMEM((bd, d), F32), pltpu.VMEM((bd, 1), F32)],
    )(st, g.reshape(1, d), bt.reshape(1, d), h, valid_e)


def _combine_sc_body(x_ref, a1_ref, w1_ref, g1_ref, b1_ref,
                     a2_ref, w2_ref, g2_ref, b2_ref, out_ref, *, n):
    x = x_ref[...]

    def sc(w_ref, g_ref, b_ref):
        s = jnp.dot(x, w_ref[...], preferred_element_type=F32)
        m = jnp.sum(s, 0, keepdims=True) / n
        d = s - m
        v = jnp.sum(d * d, 0, keepdims=True) / n
        return g_ref[...] * d * jax.lax.rsqrt(v + EPS) + b_ref[...]

    out_ref[...] = (jnp.maximum(a1_ref[...] + sc(w1_ref, g1_ref, b1_ref), 0.0)
                    + jnp.maximum(a2_ref[...] + sc(w2_ref, g2_ref, b2_ref),
                                  0.0))


def _combine_id_body(x_ref, a1_ref, a2_ref, out_ref):
    x = x_ref[...]
    out_ref[...] = (jnp.maximum(a1_ref[...] + x, 0.0)
                    + jnp.maximum(a2_ref[...] + x, 0.0))


def _combine_call(x_dst, agg1, p1, agg2, p2):
    n, d = agg1.shape
    if 'sc_W' in p1:
        body = functools.partial(_combine_sc_body, n=float(n))
        return pl.pallas_call(
            body,
            out_shape=jax.ShapeDtypeStruct((n, d), F32),
        )(x_dst, agg1, p1['sc_W'], p1['sc_g'].reshape(1, d),
          p1['sc_b'].reshape(1, d), agg2, p2['sc_W'],
          p2['sc_g'].reshape(1, d), p2['sc_b'].reshape(1, d))
    return pl.pallas_call(
        _combine_id_body,
        out_shape=jax.ShapeDtypeStruct((n, d), F32),
    )(x_dst, agg1, agg2)


def _head_body(pf_ref, sv_ref, pfb_ref, svb_ref, w1_ref, b1_ref, w2_ref,
               b2_ref, out_ref, *, npf, nsv, nev):
    ohp = (jax.lax.broadcasted_iota(jnp.int32, (nev, npf), 0)
           == pfb_ref[...]).astype(F32)
    ohs = (jax.lax.broadcasted_iota(jnp.int32, (nev, nsv), 0)
           == svb_ref[...]).astype(F32)
    sums = (jnp.dot(ohp, pf_ref[...], preferred_element_type=F32)
            + jnp.dot(ohs, sv_ref[...], preferred_element_type=F32))
    cnt = (jnp.sum(ohp, 1, keepdims=True) + jnp.sum(ohs, 1, keepdims=True))
    pooled = sums / jnp.maximum(cnt, 1.0)
    h = jnp.maximum(
        jnp.dot(pooled, w1_ref[...], preferred_element_type=F32)
        + b1_ref[...], 0.0)
    out_ref[...] = (jnp.dot(h, w2_ref[...], preferred_element_type=F32)
                    + b2_ref[...])


def _head_call(pf_f, sv_f, pf_batch, sv_batch, fc1, fc2, nev):
    npf, nsv = pf_f.shape[0], sv_f.shape[0]
    ncls = fc2['W'].shape[1]
    body = functools.partial(_head_body, npf=npf, nsv=nsv, nev=nev)
    return pl.pallas_call(
        body,
        out_shape=jax.ShapeDtypeStruct((nev, ncls), F32),
    )(pf_f, sv_f, pf_batch.reshape(1, npf), sv_batch.reshape(1, nsv),
      fc1['W'], fc1['b'].reshape(1, -1), fc2['W'], fc2['b'].reshape(1, -1))


def _edge_mlp(x_src, x_dst, idx, valid, p, k):
    nn = p['nn']
    nd = x_dst.shape[0]
    din_dst = x_dst.shape[1]
    w1 = nn[0]['W']
    a = _mm_call(x_dst, w1[:din_dst], nn[0]['b'])
    b = _mm_call(x_src, w1[din_dst:], None)
    idx_t = idx.T
    valid_e = valid.T.reshape(k * nd, 1)
    h, st = _l1_call(idx_t, a, b, k)
    h, st = _dense_call(h, st, nn[0]['g'], nn[0]['bt'], nn[1]['W'],
                        nn[1]['b'], n_prev=k * nd)
    h, st = _dense_call(h, st, nn[1]['g'], nn[1]['bt'], nn[2]['W'],
                        nn[2]['b'], n_prev=k * nd)
    return _agg_call(h, st, nn[2]['g'], nn[2]['bt'], valid_e, k, nd)


def kernel(pf_points, pf_features, sv_points, sv_features, pf_batch,
           sv_batch, params):
    ks_per_layer = (16, 7, 1, 16)
    nev = 32
    pf = _bn_call(pf_features, params['pf_bn']['g'], params['pf_bn']['bt'])
    sv = _bn_call(sv_features, params['sv_bn']['g'], params['sv_bn']['bt'])
    pf_outs, sv_outs = [], []
    for i, cp in enumerate(params['convs']):
        k_pp, k_ss, k_sp, k_ps = ks_per_layer
        pts_pf = pf_points if i == 0 else pf
        pts_sv = sv_points if i == 0 else sv
        idx_pp, v_pp = _knn_call(pts_pf, pts_pf, pf_batch, pf_batch, k_pp)
        idx_ss, v_ss = _knn_call(pts_sv, pts_sv, sv_batch, sv_batch, k_ss)
        idx_sp, v_sp = _knn_call(pts_pf, pts_sv, pf_batch, sv_batch, k_sp)
        idx_ps, v_ps = _knn_call(pts_sv, pts_pf, sv_batch, pf_batch, k_ps)
        agg_pp = _edge_mlp(pf, pf, idx_pp, v_pp, cp['pp'], k_pp)
        agg_sp = _edge_mlp(sv, pf, idx_sp, v_sp, cp['sp'], k_sp)
        agg_ss = _edge_mlp(sv, sv, idx_ss, v_ss, cp['ss'], k_ss)
        agg_ps = _edge_mlp(pf, sv, idx_ps, v_ps, cp['ps'], k_ps)
        pf_new = _combine_call(pf, agg_pp, cp['pp'], agg_sp, cp['sp'])
        sv_new = _combine_call(sv, agg_ss, cp['ss'], agg_ps, cp['ps'])
        pf, sv = pf_new, sv_new
        pf_outs.append(pf)
        sv_outs.append(sv)
    pf_cat = jnp.concatenate(pf_outs, axis=-1)
    sv_cat = jnp.concatenate(sv_outs, axis=-1)
    pf_f = _fusion_call(pf_cat, params['pf_fusion']['W'],
                        params['pf_fusion']['g'], params['pf_fusion']['bt'])
    sv_f = _fusion_call(sv_cat, params['sv_fusion']['W'],
                        params['sv_fusion']['g'], params['sv_fusion']['bt'])
    return _head_call(pf_f, sv_f, pf_batch, sv_batch, params['fc1'],
                      params['fc2'], nev)


# speedup vs baseline: 4.6393x; 1.0944x over previous
import functools

import jax
import jax.numpy as jnp
from jax.experimental import pallas as pl
from jax.experimental.pallas import tpu as pltpu
from jax.experimental.pallas import tpu_sc as plsc

F32 = jnp.float32
EPS = 1e-5


def _pick_div(n, cap):
    best = None
    for d in range(1, min(n, cap) + 1):
        if n % d == 0 and d % 8 == 0:
            best = d
    if best is None:
        best = n if n <= cap else 8
    return best


def _knn_body(dstx_ref, srcx_ref, srcn_ref, dstb_ref, srcb_ref, idx_ref,
              valid_ref, *, k, ns, bd):
    dst = dstx_ref[...]
    src = srcx_ref[...]
    dstn = jnp.sum(dst * dst, axis=1, keepdims=True)
    dot = jax.lax.dot_general(dst, src, (((1,), (1,)), ((), ())),
                              preferred_element_type=F32)
    d2 = (dstn - 2.0 * dot) + srcn_ref[...]
    d2 = jnp.where(dstb_ref[...] != srcb_ref[...], jnp.float32(1e18), d2)
    col = jax.lax.broadcasted_iota(jnp.int32, (bd, ns), 1)
    idx_cols, val_cols = [], []
    for t in range(k):
        m = jnp.min(d2, axis=1, keepdims=True)
        am = jnp.min(jnp.where(d2 <= m, col, ns), axis=1, keepdims=True)
        idx_cols.append(am)
        val_cols.append((m < 1e17).astype(F32))
        if t + 1 < k:
            d2 = jnp.where(col == am, jnp.float32(3e18), d2)
    idx_ref[...] = jnp.concatenate(idx_cols, axis=1) if k > 1 else idx_cols[0]
    valid_ref[...] = (jnp.concatenate(val_cols, axis=1) if k > 1
                      else val_cols[0])


def _rownorm_body(x_ref, y_ref):
    x = x_ref[...]
    y_ref[...] = jnp.sum(x * x, axis=1, keepdims=True)


def _rownorm_call(x):
    n = x.shape[0]
    return pl.pallas_call(
        _rownorm_body,
        out_shape=jax.ShapeDtypeStruct((n, 1), F32),
    )(x)


def _knn_call(dst_x, src_x, dst_b, src_b, k):
    nd, dim = dst_x.shape
    ns = src_x.shape[0]
    bd = _pick_div(nd, 512)
    grid = (nd // bd,)
    srcn_row = _rownorm_call(src_x).reshape(1, ns)
    body = functools.partial(_knn_body, k=k, ns=ns, bd=bd)
    idx, valid = pl.pallas_call(
        body,
        grid=grid,
        in_specs=[pl.BlockSpec((bd, dim), lambda i: (i, 0)),
                  pl.BlockSpec((ns, dim), lambda i: (0, 0)),
                  pl.BlockSpec((1, ns), lambda i: (0, 0)),
                  pl.BlockSpec((bd, 1), lambda i: (i, 0)),
                  pl.BlockSpec((1, ns), lambda i: (0, 0))],
        out_specs=[pl.BlockSpec((bd, k), lambda i: (i, 0)),
                   pl.BlockSpec((bd, k), lambda i: (i, 0))],
        out_shape=[jax.ShapeDtypeStruct((nd, k), jnp.int32),
                   jax.ShapeDtypeStruct((nd, k), F32)],
    )(dst_x, src_x, srcn_row, dst_b.reshape(nd, 1), src_b.reshape(1, ns))
    return idx, valid


def _mm_body(x_ref, w_ref, b_ref, y_ref):
    y_ref[...] = (jnp.dot(x_ref[...], w_ref[...], preferred_element_type=F32)
                  + b_ref[...])


def _mm_call(x, w, b):
    n, dout = x.shape[0], w.shape[1]
    if b is None:
        b = jnp.zeros((1, dout), F32)
    else:
        b = b.reshape(1, dout)
    return pl.pallas_call(
        _mm_body,
        out_shape=jax.ShapeDtypeStruct((n, dout), F32),
    )(x, w, b)


def _bn_body(x_ref, g_ref, bt_ref, y_ref, *, n):
    x = x_ref[...]
    m = jnp.sum(x, 0, keepdims=True) / n
    d = x - m
    v = jnp.sum(d * d, 0, keepdims=True) / n
    y_ref[...] = g_ref[...] * d * jax.lax.rsqrt(v + EPS) + bt_ref[...]


def _bn_call(x, g, bt):
    n, dim = x.shape
    return pl.pallas_call(
        functools.partial(_bn_body, n=float(n)),
        out_shape=jax.ShapeDtypeStruct((n, dim), F32),
    )(x, g.reshape(1, dim), bt.reshape(1, dim))


def _fusion_body(x_ref, w_ref, g_ref, bt_ref, y_ref, *, n):
    z = jnp.dot(x_ref[...], w_ref[...], preferred_element_type=F32)
    m = jnp.sum(z, 0, keepdims=True) / n
    d = z - m
    v = jnp.sum(d * d, 0, keepdims=True) / n
    y_ref[...] = jnp.maximum(
        g_ref[...] * d * jax.lax.rsqrt(v + EPS) + bt_ref[...], 0.0)


def _fusion_call(x, w, g, bt):
    n = x.shape[0]
    dout = w.shape[1]
    return pl.pallas_call(
        functools.partial(_fusion_body, n=float(n)),
        out_shape=jax.ShapeDtypeStruct((n, dout), F32),
    )(x, w, g.reshape(1, dout), bt.reshape(1, dout))


def _sc_gather(tables, idx_flats):
    info = plsc.get_sparse_core_info()
    nc, nsub = info.num_cores, info.num_subcores
    nw = nc * nsub
    d1 = tables[0].shape[1]
    ntab = len(tables)
    metas = []
    for ix in idx_flats:
        n_w = ix.shape[0] // nw
        c = n_w if n_w <= 128 else max(
            d for d in range(8, 129, 8) if n_w % d == 0)
        metas.append((n_w, c, n_w // c))
    mesh = plsc.VectorSubcoreMesh(core_axis_name="c", subcore_axis_name="s")
    out_type = [jax.ShapeDtypeStruct((ix.shape[0], d1), F32)
                for ix in idx_flats]
    scratch = []
    for (n_w, c, nch) in metas:
        scratch.append(pltpu.VMEM((c,), jnp.int32))
        scratch.append(pltpu.VMEM((c, d1), F32))
    scratch.append(pltpu.SemaphoreType.DMA)

    def body(*refs):
        tabs = refs[:ntab]
        idxs = refs[ntab:2 * ntab]
        outs = refs[2 * ntab:3 * ntab]
        scr = refs[3 * ntab:-1]
        sem = refs[-1]
        wid = jax.lax.axis_index("s") * nc + jax.lax.axis_index("c")
        for g in range(ntab):
            n_w, c, nch = metas[g]
            idx_v, rows_v = scr[2 * g], scr[2 * g + 1]
            base = wid * n_w
            for ch in range(nch):
                off = base + ch * c
                pltpu.sync_copy(idxs[g].at[pl.ds(off, c)], idx_v)
                pltpu.async_copy(tabs[g].at[idx_v], rows_v, sem).wait()
                pltpu.sync_copy(rows_v, outs[g].at[pl.ds(off, c)])

    fn = pl.kernel(body, mesh=mesh, out_type=out_type,
                   scratch_types=scratch)
    return fn(*tables, *idx_flats)


def _l1stats_body(gath_ref, a_ref, st_ref, *, d1):
    h = gath_ref[...] + a_ref[...]
    s = jnp.concatenate(
        [jnp.sum(h, 0, keepdims=True), jnp.sum(h * h, 0, keepdims=True),
         jnp.zeros((6, d1), F32)], axis=0)

    @pl.when((pl.program_id(0) == 0) & (pl.program_id(1) == 0))
    def _():
        st_ref[...] = jnp.zeros_like(st_ref)

    st_ref[...] += s


def _l1stats_call(gath, a, k):
    nd, d1 = a.shape
    bd = _pick_div(nd, 512)
    gi = nd // bd
    body = functools.partial(_l1stats_body, d1=d1)
    return pl.pallas_call(
        body,
        grid=(k, gi),
        in_specs=[pl.BlockSpec((bd, d1), lambda j, i: (j * gi + i, 0)),
                  pl.BlockSpec((bd, d1), lambda j, i: (i, 0))],
        out_specs=pl.BlockSpec((8, d1), lambda j, i: (0, 0)),
        out_shape=jax.ShapeDtypeStruct((8, d1), F32),
    )(gath, a)


def _dense_pair_body(stin_ref, gath_ref, a_ref, g_ref, bt_ref, w_ref, b_ref,
                     y_ref, st_ref, *, n_prev, dout):
    st = stin_ref[...]
    mean = st[0:1, :] * (1.0 / n_prev)
    var = jnp.maximum(st[1:2, :] * (1.0 / n_prev) - mean * mean, 0.0)
    scale = g_ref[...] * jax.lax.rsqrt(var + EPS)
    shift = bt_ref[...] - scale * mean
    x = gath_ref[...] + a_ref[...]
    xn = jnp.maximum(x * scale + shift, 0.0)
    y = (jnp.dot(xn, w_ref[...], preferred_element_type=F32) + b_ref[...])
    y_ref[...] = y
    s = jnp.concatenate(
        [jnp.sum(y, 0, keepdims=True), jnp.sum(y * y, 0, keepdims=True),
         jnp.zeros((6, dout), F32)], axis=0)

    @pl.when((pl.program_id(0) == 0) & (pl.program_id(1) == 0))
    def _():
        st_ref[...] = jnp.zeros_like(st_ref)

    st_ref[...] += s


def _dense_pair_call(gath, a, st, g, bt, w, b, k, n_prev):
    nd, din = a.shape
    dout = w.shape[1]
    bd = _pick_div(nd, 512)
    gi = nd // bd
    body = functools.partial(_dense_pair_body, n_prev=float(n_prev),
                             dout=dout)
    y, st_out = pl.pallas_call(
        body,
        grid=(k, gi),
        in_specs=[pl.BlockSpec((8, din), lambda j, i: (0, 0)),
                  pl.BlockSpec((bd, din), lambda j, i: (j * gi + i, 0)),
                  pl.BlockSpec((bd, din), lambda j, i: (i, 0)),
                  pl.BlockSpec((1, din), lambda j, i: (0, 0)),
                  pl.BlockSpec((1, din), lambda j, i: (0, 0)),
                  pl.BlockSpec((din, dout), lambda j, i: (0, 0)),
                  pl.BlockSpec((1, dout), lambda j, i: (0, 0))],
        out_specs=[pl.BlockSpec((bd, dout), lambda j, i: (j * gi + i, 0)),
                   pl.BlockSpec((8, dout), lambda j, i: (0, 0))],
        out_shape=[jax.ShapeDtypeStruct((k * nd, dout), F32),
                   jax.ShapeDtypeStruct((8, dout), F32)],
    )(st, gath, a, g.reshape(1, din), bt.reshape(1, din), w,
      b.reshape(1, dout))
    return y, st_out


def _dense_body(stin_ref, x_ref, g_ref, bt_ref, w_ref, b_ref, y_ref, st_ref,
                *, n_prev, dout):
    st = stin_ref[...]
    mean = st[0:1, :] * (1.0 / n_prev)
    var = jnp.maximum(st[1:2, :] * (1.0 / n_prev) - mean * mean, 0.0)
    scale = g_ref[...] * jax.lax.rsqrt(var + EPS)
    shift = bt_ref[...] - scale * mean
    xn = jnp.maximum(x_ref[...] * scale + shift, 0.0)
    y = (jnp.dot(xn, w_ref[...], preferred_element_type=F32) + b_ref[...])
    y_ref[...] = y
    s = jnp.concatenate(
        [jnp.sum(y, 0, keepdims=True), jnp.sum(y * y, 0, keepdims=True),
         jnp.zeros((6, dout), F32)], axis=0)

    @pl.when(pl.program_id(0) == 0)
    def _():
        st_ref[...] = jnp.zeros_like(st_ref)

    st_ref[...] += s


def _dense_call(h, st, g, bt, w, b, n_prev):
    ne, din = h.shape
    dout = w.shape[1]
    be = _pick_div(ne, 2048)
    body = functools.partial(_dense_body, n_prev=float(n_prev), dout=dout)
    y, st_out = pl.pallas_call(
        body,
        grid=(ne // be,),
        in_specs=[pl.BlockSpec((8, din), lambda i: (0, 0)),
                  pl.BlockSpec((be, din), lambda i: (i, 0)),
                  pl.BlockSpec((1, din), lambda i: (0, 0)),
                  pl.BlockSpec((1, din), lambda i: (0, 0)),
                  pl.BlockSpec((din, dout), lambda i: (0, 0)),
                  pl.BlockSpec((1, dout), lambda i: (0, 0))],
        out_specs=[pl.BlockSpec((be, dout), lambda i: (i, 0)),
                   pl.BlockSpec((8, dout), lambda i: (0, 0))],
        out_shape=[jax.ShapeDtypeStruct((ne, dout), F32),
                   jax.ShapeDtypeStruct((8, dout), F32)],
    )(st, h, g.reshape(1, din), bt.reshape(1, din), w, b.reshape(1, dout))
    return y, st_out


def _agg_body(stin_ref, g_ref, bt_ref, h_ref, v_ref, out_ref, acc_ref,
              cnt_ref, *, n_e, k):
    j = pl.program_id(1)
    st = stin_ref[...]
    mean = st[0:1, :] * (1.0 / n_e)
    var = jnp.maximum(st[1:2, :] * (1.0 / n_e) - mean * mean, 0.0)
    scale = g_ref[...] * jax.lax.rsqrt(var + EPS)
    shift = bt_ref[...] - scale * mean
    m = jnp.maximum(h_ref[...] * scale + shift, 0.0)
    w = v_ref[...]

    @pl.when(j == 0)
    def _():
        acc_ref[...] = jnp.zeros_like(acc_ref)
        cnt_ref[...] = jnp.zeros_like(cnt_ref)

    acc_ref[...] += m * w
    cnt_ref[...] += w

    @pl.when(j == k - 1)
    def _():
        out_ref[...] = acc_ref[...] / jnp.maximum(cnt_ref[...], 1.0)


def _agg_call(h, st, g, bt, valid_e, k, nd):
    ne, d = h.shape
    bd = _pick_div(nd, 1024)
    gi = nd // bd
    body = functools.partial(_agg_body, n_e=float(ne), k=k)
    return pl.pallas_call(
        body,
        grid=(gi, k),
        in_specs=[pl.BlockSpec((8, d), lambda i, j: (0, 0)),
                  pl.BlockSpec((1, d), lambda i, j: (0, 0)),
                  pl.BlockSpec((1, d), lambda i, j: (0, 0)),
                  pl.BlockSpec((bd, d), lambda i, j: (j * gi + i, 0)),
                  pl.BlockSpec((bd, 1), lambda i, j: (j * gi + i, 0))],
        out_specs=pl.BlockSpec((bd, d), lambda i, j: (i, 0)),
        out_shape=jax.ShapeDtypeStruct((nd, d), F32),
        scratch_shapes=[pltpu.VMEM((bd, d), F32), pltpu.VMEM((bd, 1), F32)],
    )(st, g.reshape(1, d), bt.reshape(1, d), h, valid_e)


def _combine_sc_body(x_ref, a1_ref, w1_ref, g1_ref, b1_ref,
                     a2_ref, w2_ref, g2_ref, b2_ref, out_ref, *, n):
    x = x_ref[...]

    def sc(w_ref, g_ref, b_ref):
        s = jnp.dot(x, w_ref[...], preferred_element_type=F32)
        m = jnp.sum(s, 0, keepdims=True) / n
        d = s - m
        v = jnp.sum(d * d, 0, keepdims=True) / n
        return g_ref[...] * d * jax.lax.rsqrt(v + EPS) + b_ref[...]

    out_ref[...] = (jnp.maximum(a1_ref[...] + sc(w1_ref, g1_ref, b1_ref), 0.0)
                    + jnp.maximum(a2_ref[...] + sc(w2_ref, g2_ref, b2_ref),
                                  0.0))


def _combine_id_body(x_ref, a1_ref, a2_ref, out_ref):
    x = x_ref[...]
    out_ref[...] = (jnp.maximum(a1_ref[...] + x, 0.0)
                    + jnp.maximum(a2_ref[...] + x, 0.0))


def _combine_call(x_dst, agg1, p1, agg2, p2):
    n, d = agg1.shape
    if 'sc_W' in p1:
        body = functools.partial(_combine_sc_body, n=float(n))
        return pl.pallas_call(
            body,
            out_shape=jax.ShapeDtypeStruct((n, d), F32),
        )(x_dst, agg1, p1['sc_W'], p1['sc_g'].reshape(1, d),
          p1['sc_b'].reshape(1, d), agg2, p2['sc_W'],
          p2['sc_g'].reshape(1, d), p2['sc_b'].reshape(1, d))
    return pl.pallas_call(
        _combine_id_body,
        out_shape=jax.ShapeDtypeStruct((n, d), F32),
    )(x_dst, agg1, agg2)


def _head_body(pf_ref, sv_ref, pfb_ref, svb_ref, w1_ref, b1_ref, w2_ref,
               b2_ref, out_ref, *, npf, nsv, nev):
    ohp = (jax.lax.broadcasted_iota(jnp.int32, (nev, npf), 0)
           == pfb_ref[...]).astype(F32)
    ohs = (jax.lax.broadcasted_iota(jnp.int32, (nev, nsv), 0)
           == svb_ref[...]).astype(F32)
    sums = (jnp.dot(ohp, pf_ref[...], preferred_element_type=F32)
            + jnp.dot(ohs, sv_ref[...], preferred_element_type=F32))
    cnt = (jnp.sum(ohp, 1, keepdims=True) + jnp.sum(ohs, 1, keepdims=True))
    pooled = sums / jnp.maximum(cnt, 1.0)
    h = jnp.maximum(
        jnp.dot(pooled, w1_ref[...], preferred_element_type=F32)
        + b1_ref[...], 0.0)
    out_ref[...] = (jnp.dot(h, w2_ref[...], preferred_element_type=F32)
                    + b2_ref[...])


def _head_call(pf_f, sv_f, pf_batch, sv_batch, fc1, fc2, nev):
    npf, nsv = pf_f.shape[0], sv_f.shape[0]
    ncls = fc2['W'].shape[1]
    body = functools.partial(_head_body, npf=npf, nsv=nsv, nev=nev)
    return pl.pallas_call(
        body,
        out_shape=jax.ShapeDtypeStruct((nev, ncls), F32),
    )(pf_f, sv_f, pf_batch.reshape(1, npf), sv_batch.reshape(1, nsv),
      fc1['W'], fc1['b'].reshape(1, -1), fc2['W'], fc2['b'].reshape(1, -1))


def _edge_prep(x_src, x_dst, idx, p, k):
    nn = p['nn']
    din_dst = x_dst.shape[1]
    w1 = nn[0]['W']
    d1 = w1.shape[1]
    dpad = 128 if d1 < 128 else d1
    wa, wb, b1 = w1[:din_dst], w1[din_dst:], nn[0]['b']
    if dpad != d1:
        wa = jnp.pad(wa, ((0, 0), (0, dpad - d1)))
        wb = jnp.pad(wb, ((0, 0), (0, dpad - d1)))
        b1 = jnp.pad(b1, (0, dpad - d1))
    a = _mm_call(x_dst, wa, b1)
    b = _mm_call(x_src, wb, None)
    ne = k * x_dst.shape[0]
    npad = -(-ne // 256) * 256
    flat = idx.T.reshape(ne)
    if npad != ne:
        flat = jnp.concatenate([flat, jnp.zeros((npad - ne,), jnp.int32)])
    return a, b, flat


def _edge_mlp(gath, a, valid, p, k):
    nn = p['nn']
    nd = a.shape[0]
    ne = k * nd
    d1 = nn[0]['W'].shape[1]
    dpad = a.shape[1]
    g1, bt1, w2 = nn[0]['g'], nn[0]['bt'], nn[1]['W']
    if dpad != d1:
        g1 = jnp.pad(g1, (0, dpad - d1))
        bt1 = jnp.pad(bt1, (0, dpad - d1))
        w2 = jnp.pad(w2, ((0, dpad - d1), (0, 0)))
    valid_e = valid.T.reshape(ne, 1)
    st = _l1stats_call(gath, a, k)
    h, st = _dense_pair_call(gath, a, st, g1, bt1, w2, nn[1]['b'], k,
                             n_prev=ne)
    h, st = _dense_call(h, st, nn[1]['g'], nn[1]['bt'], nn[2]['W'],
                        nn[2]['b'], n_prev=ne)
    return _agg_call(h, st, nn[2]['g'], nn[2]['bt'], valid_e, k, nd)


def kernel(pf_points, pf_features, sv_points, sv_features, pf_batch,
           sv_batch, params):
    ks_per_layer = (16, 7, 1, 16)
    nev = 32
    pf = _bn_call(pf_features, params['pf_bn']['g'], params['pf_bn']['bt'])
    sv = _bn_call(sv_features, params['sv_bn']['g'], params['sv_bn']['bt'])
    pf_outs, sv_outs = [], []
    for i, cp in enumerate(params['convs']):
        k_pp, k_ss, k_sp, k_ps = ks_per_layer
        pts_pf = pf_points if i == 0 else pf
        pts_sv = sv_points if i == 0 else sv
        idx_pp, v_pp = _knn_call(pts_pf, pts_pf, pf_batch, pf_batch, k_pp)
        idx_ss, v_ss = _knn_call(pts_sv, pts_sv, sv_batch, sv_batch, k_ss)
        idx_sp, v_sp = _knn_call(pts_pf, pts_sv, pf_batch, sv_batch, k_sp)
        idx_ps, v_ps = _knn_call(pts_sv, pts_pf, sv_batch, pf_batch, k_ps)
        specs = [(pf, pf, idx_pp, v_pp, cp['pp'], k_pp),
                 (sv, pf, idx_sp, v_sp, cp['sp'], k_sp),
                 (sv, sv, idx_ss, v_ss, cp['ss'], k_ss),
                 (pf, sv, idx_ps, v_ps, cp['ps'], k_ps)]
        preps = [_edge_prep(xs, xd, idx, p, k)
                 for (xs, xd, idx, v, p, k) in specs]
        gaths = _sc_gather([b for (_, b, _2) in preps],
                           [fl for (_, _2, fl) in preps])
        agg_pp, agg_sp, agg_ss, agg_ps = [
            _edge_mlp(gath, a, v, p, k)
            for gath, (a, _, _2), (_3, _4, _5, v, p, k)
            in zip(gaths, preps, specs)]
        pf_new = _combine_call(pf, agg_pp, cp['pp'], agg_sp, cp['sp'])
        sv_new = _combine_call(sv, agg_ss, cp['ss'], agg_ps, cp['ps'])
        pf, sv = pf_new, sv_new
        pf_outs.append(pf)
        sv_outs.append(sv)
    pf_cat = jnp.concatenate(pf_outs, axis=-1)
    sv_cat = jnp.concatenate(sv_outs, axis=-1)
    pf_f = _fusion_call(pf_cat, params['pf_fusion']['W'],
                        params['pf_fusion']['g'], params['pf_fusion']['bt'])
    sv_f = _fusion_call(sv_cat, params['sv_fusion']['W'],
                        params['sv_fusion']['g'], params['sv_fusion']['bt'])
    return _head_call(pf_f, sv_f, pf_batch, sv_batch, params['fc1'],
                      params['fc2'], nev)
